# baseline (speedup 1.0000x reference)
import functools

import jax
import jax.numpy as jnp
from jax import lax
from jax.experimental import pallas as pl
from jax.experimental.pallas import tpu as pltpu
from jax.experimental.pallas import tpu_sc as plsc

N = 10000
E = 320000
G = 64
D = 128
NCLS = 2

NC = 2
NS = 16
NW = NC * NS
EPW = E // NW
CC = 128
NBLK = 10
KPB = 8
NCH = NBLK * KPB
EPWP = NCH * CC
EPAD = NW * EPWP
NR = 10240
STRIPE = NR // NS
ZROWS = 128
NA = 10240
SA = NA // NS


def _sc_mesh():
    return plsc.VectorSubcoreMesh(
        core_axis_name="c", subcore_axis_name="s",
        num_cores=NC, num_subcores=NS)


@functools.cache
def _deg_kernel_build():
    return functools.partial(
        pl.kernel,
        out_type=jax.ShapeDtypeStruct((NC, NA), jnp.float32),
        mesh=_sc_mesh(),
        scratch_types=[
            pltpu.VMEM((NCH, CC), jnp.int32),
            pltpu.VMEM((CC,), jnp.float32),
            pltpu.VMEM_SHARED((NA,), jnp.float32),
        ],
    )(_deg_body)


def _deg_body(dst_hbm, ones_hbm, z1_hbm, out_hbm, dst_v, ones_v, acc_sh):
    c = lax.axis_index("c")
    s = lax.axis_index("s")
    w = c * NS + s

    pltpu.sync_copy(z1_hbm.at[pl.ds(s * SA, SA)],
                    acc_sh.at[pl.ds(s * SA, SA)])
    pltpu.sync_copy(ones_hbm, ones_v)
    pltpu.sync_copy(dst_hbm.at[w], dst_v)
    plsc.subcore_barrier()

    @pl.loop(0, NCH)
    def _chunks(j):
        pltpu.sync_copy(ones_v, acc_sh.at[dst_v.at[j]], add=True)

    plsc.subcore_barrier()
    pltpu.sync_copy(acc_sh.at[pl.ds(s * SA, SA)],
                    out_hbm.at[c, pl.ds(s * SA, SA)])


@functools.cache
def _agg_kernel_build():
    return functools.partial(
        pl.kernel,
        out_type=jax.ShapeDtypeStruct((NC, NR, D), jnp.float32),
        mesh=_sc_mesh(),
        scratch_types=[
            pltpu.VMEM((KPB, CC), jnp.int32),
            pltpu.VMEM((KPB, CC), jnp.int32),
            pltpu.VMEM((KPB, CC), jnp.int32),
            pltpu.VMEM((KPB, CC), jnp.int32),
            pltpu.VMEM((CC, D), jnp.float32),
            pltpu.VMEM((CC, D), jnp.float32),
            pltpu.VMEM_SHARED((NR, D), jnp.float32),
            pltpu.SemaphoreType.DMA,
            pltpu.SemaphoreType.DMA,
            pltpu.SemaphoreType.DMA,
            pltpu.SemaphoreType.DMA,
            pltpu.SemaphoreType.DMA,
        ],
    )(_agg_body)


def _agg_body(g_hbm, src_hbm, dst_hbm, z_hbm, out_hbm,
              sidx0, didx0, sidx1, didx1, rows0, rows1, acc_sh,
              gs0, gs1, ss0, ss1, ism):
    c = lax.axis_index("c")
    s = lax.axis_index("s")
    w = c * NS + s
    rows = (rows0, rows1)
    gs = (gs0, gs1)
    ss = (ss0, ss1)
    sidx = (sidx0, sidx1)
    didx = (didx0, didx1)

    pltpu.sync_copy(z_hbm.at[pl.ds(s * STRIPE, STRIPE)],
                    acc_sh.at[pl.ds(s * STRIPE, STRIPE)])
    pltpu.sync_copy(src_hbm.at[w, 0], sidx0)
    pltpu.sync_copy(dst_hbm.at[w, 0], didx0)
    plsc.subcore_barrier()
    pltpu.async_copy(g_hbm.at[sidx0.at[0]], rows0, gs0)
    pltpu.async_copy(src_hbm.at[w, 1], sidx1, ism)
    pltpu.async_copy(dst_hbm.at[w, 1], didx1, ism)

    def _chunk(it, blk_set, k):
        p = k % 2
        q = (k + 1) % 2
        oset = 1 - blk_set
        sidx_b, didx_b = sidx[blk_set], didx[blk_set]

        if k == 0:
            def _head():
                pltpu.make_async_copy(rows[1], acc_sh.at[didx_b.at[0]],
                                      ss[1]).wait()
                nxt_blk = 2 * it + blk_set + 1
                pltpu.async_copy(src_hbm.at[w, nxt_blk], sidx[oset], ism)
                pltpu.async_copy(dst_hbm.at[w, nxt_blk], didx[oset], ism)

            if blk_set == 0:
                pl.when(it > 0)(_head)
            else:
                @pl.when(it < NBLK // 2 - 1)
                def _():
                    _head()

                @pl.when(it == NBLK // 2 - 1)
                def _():
                    pltpu.make_async_copy(rows[1], acc_sh.at[didx_b.at[0]],
                                          ss[1]).wait()

        pltpu.make_async_copy(g_hbm.at[sidx_b.at[k]], rows[p], gs[p]).wait()
        pltpu.async_copy(rows[p], acc_sh.at[didx_b.at[k]], ss[p], add=True)

        if k > 0:
            pltpu.make_async_copy(rows[q], acc_sh.at[didx_b.at[k]],
                                  ss[q]).wait()

        if k < KPB - 1:
            pltpu.async_copy(g_hbm.at[sidx_b.at[k + 1]], rows[q], gs[q])
        else:
            def _tail():
                pltpu.make_async_copy(src_hbm.at[w, 0], sidx[oset],
                                      ism).wait()
                pltpu.make_async_copy(dst_hbm.at[w, 0], didx[oset],
                                      ism).wait()
                pltpu.async_copy(g_hbm.at[sidx[oset].at[0]], rows[q], gs[q])

            if blk_set == 0:
                _tail()
            else:
                pl.when(it < NBLK // 2 - 1)(_tail)

    @pl.loop(0, NBLK // 2)
    def _two_blocks(it):
        for blk_set in (0, 1):
            for k in range(KPB):
                _chunk(it, blk_set, k)

    pltpu.make_async_copy(rows1, acc_sh.at[didx1.at[0]], ss1).wait()
    plsc.subcore_barrier()
    pltpu.sync_copy(acc_sh.at[pl.ds(s * STRIPE, STRIPE)],
                    out_hbm.at[c, pl.ds(s * STRIPE, STRIPE)])


BR = 1000


def _prep_body(p0, p1, x, w1, dinv_ref, g_ref):
    deg = p0[...] + p1[...] + 1.0
    dinv = lax.rsqrt(deg)
    dinv_ref[...] = dinv
    g_ref[...] = dinv * jnp.dot(x[...], w1[...],
                                preferred_element_type=jnp.float32)


def _prep(p0, p1, x, w1):
    return pl.pallas_call(
        _prep_body,
        grid=(N // BR,),
        in_specs=[
            pl.BlockSpec((BR, 1), lambda i: (i, 0)),
            pl.BlockSpec((BR, 1), lambda i: (i, 0)),
            pl.BlockSpec((BR, D), lambda i: (i, 0)),
            pl.BlockSpec((D, D), lambda i: (0, 0)),
        ],
        out_specs=[
            pl.BlockSpec((BR, 1), lambda i: (i, 0)),
            pl.BlockSpec((BR, D), lambda i: (i, 0)),
        ],
        out_shape=[
            jax.ShapeDtypeStruct((N, 1), jnp.float32),
            jax.ShapeDtypeStruct((N, D), jnp.float32),
        ],
    )(p0, p1, x, w1)


def _layer_body(a0, a1, g, dinv, b, w, gn_ref):
    h = jnp.maximum(dinv[...] * (a0[...] + a1[...] + g[...]) + b[...], 0.0)
    gn_ref[...] = dinv[...] * jnp.dot(h, w[...],
                                      preferred_element_type=jnp.float32)


def _layer(a0, a1, g, dinv, b, w):
    return pl.pallas_call(
        _layer_body,
        grid=(N // BR,),
        in_specs=[
            pl.BlockSpec((BR, D), lambda i: (i, 0)),
            pl.BlockSpec((BR, D), lambda i: (i, 0)),
            pl.BlockSpec((BR, D), lambda i: (i, 0)),
            pl.BlockSpec((BR, 1), lambda i: (i, 0)),
            pl.BlockSpec((1, D), lambda i: (0, 0)),
            pl.BlockSpec((D, D), lambda i: (0, 0)),
        ],
        out_specs=pl.BlockSpec((BR, D), lambda i: (i, 0)),
        out_shape=jax.ShapeDtypeStruct((N, D), jnp.float32),
    )(a0, a1, g, dinv, b, w)


def _head_body(a0, a1, g, dinv, b, batch, wh, bh, out_ref, psum, cnt):
    i = pl.program_id(0)

    @pl.when(i == 0)
    def _():
        psum[...] = jnp.zeros_like(psum)
        cnt[...] = jnp.zeros_like(cnt)

    h = jnp.maximum(dinv[...] * (a0[...] + a1[...] + g[...]) + b[...], 0.0)
    sel = (batch[...] == lax.broadcasted_iota(jnp.int32, (BR, G), 1)
           ).astype(jnp.float32)
    dn = (((0,), (0,)), ((), ()))
    psum[...] += lax.dot_general(sel, h, dn,
                                 preferred_element_type=jnp.float32)
    cnt[...] += lax.dot_general(sel, jnp.ones((BR, 1), jnp.float32), dn,
                                preferred_element_type=jnp.float32)

    @pl.when(i == pl.num_programs(0) - 1)
    def _():
        pooled = psum[...] / jnp.maximum(cnt[...], 1.0)
        out_ref[...] = jnp.dot(pooled, wh[...],
                               preferred_element_type=jnp.float32) + bh[...]


def _head(a0, a1, g, dinv, b, batch, wh, bh):
    return pl.pallas_call(
        _head_body,
        grid=(N // BR,),
        in_specs=[
            pl.BlockSpec((BR, D), lambda i: (i, 0)),
            pl.BlockSpec((BR, D), lambda i: (i, 0)),
            pl.BlockSpec((BR, D), lambda i: (i, 0)),
            pl.BlockSpec((BR, 1), lambda i: (i, 0)),
            pl.BlockSpec((1, D), lambda i: (0, 0)),
            pl.BlockSpec((BR, 1), lambda i: (i, 0)),
            pl.BlockSpec((D, NCLS), lambda i: (0, 0)),
            pl.BlockSpec((1, NCLS), lambda i: (0, 0)),
        ],
        out_specs=pl.BlockSpec((G, NCLS), lambda i: (0, 0)),
        out_shape=jax.ShapeDtypeStruct((G, NCLS), jnp.float32),
        scratch_shapes=[
            pltpu.VMEM((G, D), jnp.float32),
            pltpu.VMEM((G, 1), jnp.float32),
        ],
    )(a0, a1, g, dinv, b, batch, wh, bh)


def kernel(x, edge_index, batch, W1, b1, W2, b2, W3, b3, Wh, bh):
    pad = EPAD - E
    srcp = jnp.concatenate([edge_index[0].astype(jnp.int32),
                            jnp.zeros((pad,), jnp.int32)])
    dstp = jnp.concatenate([edge_index[1].astype(jnp.int32),
                            jnp.full((pad,), N, jnp.int32)])
    src4 = srcp.reshape(NW, NBLK, KPB, CC)
    dst4 = dstp.reshape(NW, NBLK, KPB, CC)
    dst3 = dstp.reshape(NW, NCH, CC)

    ones128 = jnp.ones((CC,), jnp.float32)
    zrow = jnp.zeros((NR, D), jnp.float32)
    z1 = jnp.zeros((NA,), jnp.float32)

    degp = _deg_kernel_build()(dst3, ones128, z1)
    p0 = degp[0, :N].reshape(N, 1)
    p1 = degp[1, :N].reshape(N, 1)

    dinv, g1 = _prep(p0, p1, x, W1)
    agg = _agg_kernel_build()

    acc = agg(g1, src4, dst4, zrow)
    g2 = _layer(acc[0], acc[1], g1, dinv, b1.reshape(1, D), W2)
    acc = agg(g2, src4, dst4, zrow)
    g3 = _layer(acc[0], acc[1], g2, dinv, b2.reshape(1, D), W3)
    acc = agg(g3, src4, dst4, zrow)

    return _head(acc[0], acc[1], g3, dinv, b3.reshape(1, D),
                 batch.astype(jnp.int32).reshape(N, 1),
                 Wh, bh.reshape(1, NCLS))


# speedup vs baseline: 2.6642x; 2.6642x over previous
import functools

import jax
import jax.numpy as jnp
from jax import lax
from jax.experimental import pallas as pl
from jax.experimental.pallas import tpu as pltpu
from jax.experimental.pallas import tpu_sc as plsc

N = 10000
E = 320000
G = 64
D = 128
NCLS = 2

NC = 2
NS = 16
NW = NC * NS
EPW = E // NW
C = 80
NCH = EPW // C
NR = 10112
STRIPE = NR // NS
NA = 10240
SA = NA // NS


def _sc_mesh():
    return plsc.VectorSubcoreMesh(
        core_axis_name="c", subcore_axis_name="s",
        num_cores=NC, num_subcores=NS)


@functools.cache
def _deg_kernel_build():
    return functools.partial(
        pl.kernel,
        out_type=jax.ShapeDtypeStruct((NC, NA), jnp.float32),
        mesh=_sc_mesh(),
        scratch_types=[
            pltpu.VMEM((NCH, C), jnp.int32),
            pltpu.VMEM((128,), jnp.float32),
            pltpu.VMEM_SHARED((NA,), jnp.float32),
        ],
    )(_deg_body)


def _deg_body(dst_hbm, ones_hbm, z1_hbm, out_hbm, dst_v, ones_v, acc_sh):
    c = lax.axis_index("c")
    s = lax.axis_index("s")
    w = c * NS + s

    pltpu.sync_copy(z1_hbm.at[pl.ds(s * SA, SA)],
                    acc_sh.at[pl.ds(s * SA, SA)])
    pltpu.sync_copy(ones_hbm, ones_v)
    pltpu.sync_copy(dst_hbm.at[w], dst_v)
    plsc.subcore_barrier()

    @pl.loop(0, NCH)
    def _chunks(j):
        pltpu.sync_copy(ones_v.at[pl.ds(0, C)], acc_sh.at[dst_v.at[j]],
                        add=True)

    plsc.subcore_barrier()
    pltpu.sync_copy(acc_sh.at[pl.ds(s * SA, SA)],
                    out_hbm.at[c, pl.ds(s * SA, SA)])


@functools.cache
def _agg_kernel_build():
    return functools.partial(
        pl.kernel,
        out_type=jax.ShapeDtypeStruct((NC, NR, D), jnp.float32),
        mesh=_sc_mesh(),
        scratch_types=[
            pltpu.VMEM((EPW,), jnp.int32),
            pltpu.VMEM((NCH, C), jnp.int32),
            pltpu.VMEM((C, D), jnp.float32),
            pltpu.VMEM((C, D), jnp.float32),
            pltpu.VMEM_SHARED((NR, D), jnp.float32),
            pltpu.SemaphoreType.DMA,
            pltpu.SemaphoreType.DMA,
        ],
    )(_agg_body)


def _agg_body(g_hbm, src_hbm, dst_hbm, z_hbm, out_hbm,
              src_v, dst_v, rows0, rows1, acc_sh, gs0, gs1):
    c = lax.axis_index("c")
    s = lax.axis_index("s")
    w = c * NS + s

    pltpu.sync_copy(z_hbm.at[pl.ds(s * STRIPE, STRIPE)],
                    acc_sh.at[pl.ds(s * STRIPE, STRIPE)])
    pltpu.sync_copy(src_hbm.at[w], src_v)
    pltpu.sync_copy(dst_hbm.at[w], dst_v)
    plsc.subcore_barrier()

    pltpu.async_copy(g_hbm.at[src_v.at[pl.ds(0, C)]], rows0, gs0)

    @pl.loop(0, NCH // 2)
    def _pair(it):
        j0 = 2 * it
        pltpu.make_async_copy(g_hbm.at[src_v.at[pl.ds(j0 * C, C)]],
                              rows0, gs0).wait()
        pltpu.async_copy(g_hbm.at[src_v.at[pl.ds((j0 + 1) * C, C)]],
                         rows1, gs1)
        pltpu.sync_copy(rows0, acc_sh.at[dst_v.at[j0]], add=True)
        pltpu.make_async_copy(g_hbm.at[src_v.at[pl.ds((j0 + 1) * C, C)]],
                              rows1, gs1).wait()
        pltpu.async_copy(g_hbm.at[src_v.at[pl.ds((j0 + 2) * C, C)]],
                         rows0, gs0)
        pltpu.sync_copy(rows1, acc_sh.at[dst_v.at[j0 + 1]], add=True)

    pltpu.make_async_copy(g_hbm.at[src_v.at[pl.ds((NCH - 1) * C, C)]],
                          rows0, gs0).wait()
    pltpu.sync_copy(rows0, acc_sh.at[dst_v.at[NCH - 1]], add=True)

    plsc.subcore_barrier()
    pltpu.sync_copy(acc_sh.at[pl.ds(s * STRIPE, STRIPE)],
                    out_hbm.at[c, pl.ds(s * STRIPE, STRIPE)])


BR = 1000


def _prep_body(p0, p1, x, w1, dinv_ref, g_ref):
    deg = p0[...] + p1[...] + 1.0
    dinv = lax.rsqrt(deg)
    dinv_ref[...] = dinv
    g_ref[...] = dinv * jnp.dot(x[...], w1[...],
                                preferred_element_type=jnp.float32)


def _prep(p0, p1, x, w1):
    return pl.pallas_call(
        _prep_body,
        grid=(N // BR,),
        in_specs=[
            pl.BlockSpec((BR, 1), lambda i: (i, 0)),
            pl.BlockSpec((BR, 1), lambda i: (i, 0)),
            pl.BlockSpec((BR, D), lambda i: (i, 0)),
            pl.BlockSpec((D, D), lambda i: (0, 0)),
        ],
        out_specs=[
            pl.BlockSpec((BR, 1), lambda i: (i, 0)),
            pl.BlockSpec((BR, D), lambda i: (i, 0)),
        ],
        out_shape=[
            jax.ShapeDtypeStruct((N, 1), jnp.float32),
            jax.ShapeDtypeStruct((N, D), jnp.float32),
        ],
    )(p0, p1, x, w1)


def _layer_body(a0, a1, g, dinv, b, w, gn_ref):
    h = jnp.maximum(dinv[...] * (a0[...] + a1[...] + g[...]) + b[...], 0.0)
    gn_ref[...] = dinv[...] * jnp.dot(h, w[...],
                                      preferred_element_type=jnp.float32)


def _layer(a0, a1, g, dinv, b, w):
    return pl.pallas_call(
        _layer_body,
        grid=(N // BR,),
        in_specs=[
            pl.BlockSpec((BR, D), lambda i: (i, 0)),
            pl.BlockSpec((BR, D), lambda i: (i, 0)),
            pl.BlockSpec((BR, D), lambda i: (i, 0)),
            pl.BlockSpec((BR, 1), lambda i: (i, 0)),
            pl.BlockSpec((1, D), lambda i: (0, 0)),
            pl.BlockSpec((D, D), lambda i: (0, 0)),
        ],
        out_specs=pl.BlockSpec((BR, D), lambda i: (i, 0)),
        out_shape=jax.ShapeDtypeStruct((N, D), jnp.float32),
    )(a0, a1, g, dinv, b, w)


def _head_body(a0, a1, g, dinv, b, batch, wh, bh, out_ref, psum, cnt):
    i = pl.program_id(0)

    @pl.when(i == 0)
    def _():
        psum[...] = jnp.zeros_like(psum)
        cnt[...] = jnp.zeros_like(cnt)

    h = jnp.maximum(dinv[...] * (a0[...] + a1[...] + g[...]) + b[...], 0.0)
    sel = (batch[...] == lax.broadcasted_iota(jnp.int32, (BR, G), 1)
           ).astype(jnp.float32)
    dn = (((0,), (0,)), ((), ()))
    psum[...] += lax.dot_general(sel, h, dn,
                                 preferred_element_type=jnp.float32)
    cnt[...] += lax.dot_general(sel, jnp.ones((BR, 1), jnp.float32), dn,
                                preferred_element_type=jnp.float32)

    @pl.when(i == pl.num_programs(0) - 1)
    def _():
        pooled = psum[...] / jnp.maximum(cnt[...], 1.0)
        out_ref[...] = jnp.dot(pooled, wh[...],
                               preferred_element_type=jnp.float32) + bh[...]


def _head(a0, a1, g, dinv, b, batch, wh, bh):
    return pl.pallas_call(
        _head_body,
        grid=(N // BR,),
        in_specs=[
            pl.BlockSpec((BR, D), lambda i: (i, 0)),
            pl.BlockSpec((BR, D), lambda i: (i, 0)),
            pl.BlockSpec((BR, D), lambda i: (i, 0)),
            pl.BlockSpec((BR, 1), lambda i: (i, 0)),
            pl.BlockSpec((1, D), lambda i: (0, 0)),
            pl.BlockSpec((BR, 1), lambda i: (i, 0)),
            pl.BlockSpec((D, NCLS), lambda i: (0, 0)),
            pl.BlockSpec((1, NCLS), lambda i: (0, 0)),
        ],
        out_specs=pl.BlockSpec((G, NCLS), lambda i: (0, 0)),
        out_shape=jax.ShapeDtypeStruct((G, NCLS), jnp.float32),
        scratch_shapes=[
            pltpu.VMEM((G, D), jnp.float32),
            pltpu.VMEM((G, 1), jnp.float32),
        ],
    )(a0, a1, g, dinv, b, batch, wh, bh)


def kernel(x, edge_index, batch, W1, b1, W2, b2, W3, b3, Wh, bh):
    src2 = edge_index[0].astype(jnp.int32).reshape(NW, EPW)
    dst3 = edge_index[1].astype(jnp.int32).reshape(NW, NCH, C)

    ones128 = jnp.ones((128,), jnp.float32)
    zrow = jnp.zeros((NR, D), jnp.float32)
    z1 = jnp.zeros((NA,), jnp.float32)

    degp = _deg_kernel_build()(dst3, ones128, z1)
    p0 = degp[0, :N].reshape(N, 1)
    p1 = degp[1, :N].reshape(N, 1)

    dinv, g1 = _prep(p0, p1, x, W1)
    agg = _agg_kernel_build()

    acc = agg(g1, src2, dst3, zrow)
    g2 = _layer(acc[0], acc[1], g1, dinv, b1.reshape(1, D), W2)
    acc = agg(g2, src2, dst3, zrow)
    g3 = _layer(acc[0], acc[1], g2, dinv, b2.reshape(1, D), W3)
    acc = agg(g3, src2, dst3, zrow)

    return _head(acc[0], acc[1], g3, dinv, b3.reshape(1, D),
                 batch.astype(jnp.int32).reshape(N, 1),
                 Wh, bh.reshape(1, NCLS))
